# Initial kernel scaffold; baseline (speedup 1.0000x reference)
import jax
import jax.numpy as jnp
from jax.experimental import pallas as pl


def kernel(ingredient_ids, table):
    raise NotImplementedError("write your pallas kernel here")


# speedup vs baseline: 2.1461x; 2.1461x over previous
import functools

import jax
import jax.numpy as jnp
from jax import lax
from jax.experimental import pallas as pl
from jax.experimental.pallas import tpu as pltpu
from jax.experimental.pallas import tpu_sc as plsc

NUM_CORES = 2
NUM_SUBCORES = 16
NUM_WORKERS = NUM_CORES * NUM_SUBCORES
LANES = 16

ROWS_PER_CHUNK = 2
NBUF = 4


def _sc_partial_sums(ids3, table, num_chunks, ids_per_chunk, L, D):
  vecs_per_row = D // LANES

  mesh = plsc.VectorSubcoreMesh(
      core_axis_name="c", subcore_axis_name="s",
      num_cores=NUM_CORES, num_subcores=NUM_SUBCORES)

  scratch = (
      [pltpu.VMEM((num_chunks, ids_per_chunk), jnp.int32)]
      + [pltpu.VMEM((ids_per_chunk, D), jnp.float32) for _ in range(NBUF)]
      + [pltpu.VMEM((L, D), jnp.float32)]
      + [pltpu.SemaphoreType.DMA for _ in range(NBUF)]
  )

  @functools.partial(
      pl.kernel,
      out_type=jax.ShapeDtypeStruct((NUM_WORKERS, L, D), jnp.float32),
      mesh=mesh,
      scratch_types=scratch,
      compiler_params=pltpu.CompilerParams(use_tc_tiling_on_sc=False),
  )
  def body(ids_hbm, table_hbm, out_hbm, *refs):
    idx_v = refs[0]
    rows = refs[1:1 + NBUF]
    acc_v = refs[1 + NBUF]
    sems = refs[2 + NBUF:2 + 2 * NBUF]

    wid = lax.axis_index("s") * NUM_CORES + lax.axis_index("c")

    pltpu.sync_copy(ids_hbm.at[wid], idx_v)

    zero = jnp.zeros((LANES,), jnp.float32)
    for r in range(L):
      for h in range(vecs_per_row):
        acc_v[r, pl.ds(h * LANES, LANES)] = zero

    for b in range(NBUF):
      pltpu.async_copy(table_hbm.at[idx_v.at[b]], rows[b], sems[b])

    def loop_body(it, carry):
      j = it * NBUF
      for b in range(NBUF):
        cur = j + b
        pltpu.make_async_copy(
            table_hbm.at[idx_v.at[cur]], rows[b], sems[b]).wait()
        for r in range(ids_per_chunk):
          ar = r % L
          for h in range(vecs_per_row):
            x = rows[b][r, pl.ds(h * LANES, LANES)]
            plsc.addupdate(acc_v.at[ar, pl.ds(h * LANES, LANES)], x)
        nxt = cur + NBUF

        @pl.when(nxt < num_chunks)
        def _():
          pltpu.async_copy(table_hbm.at[idx_v.at[nxt]], rows[b], sems[b])
      return carry

    lax.fori_loop(0, num_chunks // NBUF, loop_body, 0, unroll=False)

    pltpu.sync_copy(acc_v, out_hbm.at[wid])

  return body(ids3, table)


def _tc_combine(partials, L, D):
  def body(x_ref, o_ref):
    o_ref[...] = jnp.sum(x_ref[...], axis=0)

  return pl.pallas_call(
      body,
      out_shape=jax.ShapeDtypeStruct((L, D), jnp.float32),
  )(partials)


def kernel(ingredient_ids, table):
  B, L = ingredient_ids.shape
  V, D = table.shape
  ids = ingredient_ids.astype(jnp.int32)

  rows_per_worker = B // NUM_WORKERS
  ids_per_chunk = ROWS_PER_CHUNK * L
  num_chunks = rows_per_worker // ROWS_PER_CHUNK
  assert B % NUM_WORKERS == 0
  assert rows_per_worker % ROWS_PER_CHUNK == 0
  assert num_chunks % NBUF == 0
  assert ids_per_chunk <= 128 and D % LANES == 0

  ids3 = ids.reshape(NUM_WORKERS, num_chunks, ids_per_chunk)
  partials = _sc_partial_sums(ids3, table, num_chunks, ids_per_chunk, L, D)
  return _tc_combine(partials, L, D)
